# baseline (speedup 1.0000x reference)
import jax
import jax.numpy as jnp
from jax import lax
from jax.experimental import pallas as pl
from jax.experimental.pallas import tpu as pltpu

MAXD = 384.0
INTERVAL = 381.0 / 127.0
WD = 0.9
WL = 0.1

BH = 16


def _tc_body(pred_ref, gt_ref, valid_ref, logits_ref, l1_ref, ce_ref, cnt_ref):
    b = pl.program_id(0)
    i = pl.program_id(1)

    @pl.when((b == 0) & (i == 0))
    def _init():
        l1_ref[0] = 0.0
        ce_ref[0] = 0.0
        cnt_ref[0] = 0.0

    gt = gt_ref[0]
    pred = pred_ref[0]
    vmask = valid_ref[0]
    mask = jnp.where(gt < MAXD, vmask, 0.0)

    l1 = jnp.abs(pred - gt) * mask

    labels = jnp.clip(gt, 0.0, 381.0) / INTERVAL

    m = jnp.max(logits_ref[0], axis=0)

    CH = 1
    C = 128
    s = jnp.zeros(m.shape, jnp.float32)
    g = jnp.zeros(m.shape, jnp.float32)
    for j in range(0, C, CH):
        xc = logits_ref[0, j:j + CH]
        s = s + jnp.sum(jnp.exp(xc - m[None]), axis=0)
        cf = (lax.broadcasted_iota(jnp.int32, (CH, 1, 1), 0) + j).astype(jnp.float32)
        w = jnp.maximum(1.0 - jnp.abs(labels[None] - cf), 0.0)
        g = g + jnp.sum(xc * w, axis=0)
    lse = m + jnp.log(s)

    ce = (lse - g) * mask

    l1_ref[0] += jnp.sum(l1)
    ce_ref[0] += jnp.sum(ce)
    cnt_ref[0] += jnp.sum(mask)


def kernel(pred_disp, disp_logits, gt_disp, valid):
    B, C, H, W = disp_logits.shape
    pred_disp = pred_disp.astype(jnp.float32)
    gt_disp = gt_disp.astype(jnp.float32)
    validf = valid.astype(jnp.float32)
    logits = disp_logits.astype(jnp.float32)
    nb = H // BH

    l1_sum, ce_sum, cnt = pl.pallas_call(
        _tc_body,
        grid=(B, nb),
        in_specs=[
            pl.BlockSpec((1, BH, W), lambda b, i: (b, i, 0)),
            pl.BlockSpec((1, BH, W), lambda b, i: (b, i, 0)),
            pl.BlockSpec((1, BH, W), lambda b, i: (b, i, 0)),
            pl.BlockSpec((1, C, BH, W), lambda b, i: (b, 0, i, 0)),
        ],
        out_specs=[
            pl.BlockSpec(memory_space=pltpu.SMEM),
            pl.BlockSpec(memory_space=pltpu.SMEM),
            pl.BlockSpec(memory_space=pltpu.SMEM),
        ],
        out_shape=[jax.ShapeDtypeStruct((1,), jnp.float32)] * 3,
    )(pred_disp, gt_disp, validf, logits)

    denom = cnt[0] + 1e-6
    loss_disp = l1_sum[0] / denom
    loss_logits = ce_sum[0] / denom
    objective = WD * loss_disp + WL * loss_logits
    return objective, loss_disp, loss_logits


# speedup vs baseline: 1.1558x; 1.1558x over previous
import jax
import jax.numpy as jnp
from jax import lax
from jax.experimental import pallas as pl
from jax.experimental.pallas import tpu as pltpu

MAXD = 384.0
INTERVAL = 381.0 / 127.0
WD = 0.9
WL = 0.1

BH = 32
C = 128
CHALF = C // 2


def _tc_body(pred_ref, gt_ref, valid_ref, la_ref, lb_ref, l1_ref, ce_ref, cnt_ref):
    b = pl.program_id(0)
    i = pl.program_id(1)

    @pl.when((b == 0) & (i == 0))
    def _init():
        l1_ref[0] = 0.0
        ce_ref[0] = 0.0
        cnt_ref[0] = 0.0

    gt = gt_ref[0]
    pred = pred_ref[0]
    vmask = valid_ref[0]
    mask = jnp.where(gt < MAXD, vmask, 0.0)

    l1 = jnp.abs(pred - gt) * mask

    labels = jnp.clip(gt, 0.0, 381.0) / INTERVAL

    m = jnp.maximum(jnp.max(la_ref[0], axis=0), jnp.max(lb_ref[0], axis=0))

    s = jnp.zeros(m.shape, jnp.float32)
    g = jnp.zeros(m.shape, jnp.float32)
    for j in range(C):
        ref = la_ref if j < CHALF else lb_ref
        xc = ref[0, j % CHALF]
        s = s + jnp.exp(xc - m)
        w = jnp.maximum(1.0 - jnp.abs(labels - float(j)), 0.0)
        g = g + xc * w
    lse = m + jnp.log(s)

    ce = (lse - g) * mask

    l1_ref[0] += jnp.sum(l1)
    ce_ref[0] += jnp.sum(ce)
    cnt_ref[0] += jnp.sum(mask)


def kernel(pred_disp, disp_logits, gt_disp, valid):
    B, Cc, H, W = disp_logits.shape
    pred_disp = pred_disp.astype(jnp.float32)
    gt_disp = gt_disp.astype(jnp.float32)
    validf = valid.astype(jnp.float32)
    logits = disp_logits.astype(jnp.float32)
    nb = H // BH

    l1_sum, ce_sum, cnt = pl.pallas_call(
        _tc_body,
        grid=(B, nb),
        in_specs=[
            pl.BlockSpec((1, BH, W), lambda b, i: (b, i, 0)),
            pl.BlockSpec((1, BH, W), lambda b, i: (b, i, 0)),
            pl.BlockSpec((1, BH, W), lambda b, i: (b, i, 0)),
            pl.BlockSpec((1, CHALF, BH, W), lambda b, i: (b, 0, i, 0)),
            pl.BlockSpec((1, CHALF, BH, W), lambda b, i: (b, 1, i, 0)),
        ],
        out_specs=[
            pl.BlockSpec(memory_space=pltpu.SMEM),
            pl.BlockSpec(memory_space=pltpu.SMEM),
            pl.BlockSpec(memory_space=pltpu.SMEM),
        ],
        out_shape=[jax.ShapeDtypeStruct((1,), jnp.float32)] * 3,
    )(pred_disp, gt_disp, validf, logits, logits)

    denom = cnt[0] + 1e-6
    loss_disp = l1_sum[0] / denom
    loss_logits = ce_sum[0] / denom
    objective = WD * loss_disp + WL * loss_logits
    return objective, loss_disp, loss_logits


# speedup vs baseline: 1.2411x; 1.0738x over previous
import jax
import jax.numpy as jnp
from jax import lax
from jax.experimental import pallas as pl
from jax.experimental.pallas import tpu as pltpu

MAXD = 384.0
INTERVAL = 381.0 / 127.0
WD = 0.9
WL = 0.1

BH = 32
C = 128


def _tc_body(pred_ref, gt_ref, valid_ref, logits_ref, l1_ref, ce_ref, cnt_ref):
    b = pl.program_id(0)
    i = pl.program_id(1)

    @pl.when((b == 0) & (i == 0))
    def _init():
        l1_ref[0] = 0.0
        ce_ref[0] = 0.0
        cnt_ref[0] = 0.0

    gt = gt_ref[0]
    pred = pred_ref[0]
    vmask = valid_ref[0]
    mask = jnp.where(gt < MAXD, vmask, 0.0)

    l1 = jnp.abs(pred - gt) * mask

    labels = jnp.clip(gt, 0.0, 381.0) / INTERVAL

    m = jnp.max(logits_ref[0], axis=0)

    s = jnp.zeros(m.shape, jnp.float32)
    g = jnp.zeros(m.shape, jnp.float32)
    for j in range(C):
        xc = logits_ref[0, j]
        s = s + jnp.exp(xc - m)
        w = jnp.maximum(1.0 - jnp.abs(labels - float(j)), 0.0)
        g = g + xc * w
    lse = m + jnp.log(s)

    ce = (lse - g) * mask

    l1_ref[0] += jnp.sum(l1)
    ce_ref[0] += jnp.sum(ce)
    cnt_ref[0] += jnp.sum(mask)


def kernel(pred_disp, disp_logits, gt_disp, valid):
    B, Cc, H, W = disp_logits.shape
    pred_disp = pred_disp.astype(jnp.float32)
    gt_disp = gt_disp.astype(jnp.float32)
    validf = valid.astype(jnp.float32)
    logits = disp_logits.astype(jnp.float32)
    nb = H // BH

    l1_sum, ce_sum, cnt = pl.pallas_call(
        _tc_body,
        grid=(B, nb),
        in_specs=[
            pl.BlockSpec((1, BH, W), lambda b, i: (b, i, 0)),
            pl.BlockSpec((1, BH, W), lambda b, i: (b, i, 0)),
            pl.BlockSpec((1, BH, W), lambda b, i: (b, i, 0)),
            pl.BlockSpec((1, Cc, BH, W), lambda b, i: (b, 0, i, 0)),
        ],
        out_specs=[
            pl.BlockSpec(memory_space=pltpu.SMEM),
            pl.BlockSpec(memory_space=pltpu.SMEM),
            pl.BlockSpec(memory_space=pltpu.SMEM),
        ],
        out_shape=[jax.ShapeDtypeStruct((1,), jnp.float32)] * 3,
    )(pred_disp, gt_disp, validf, logits)

    denom = cnt[0] + 1e-6
    loss_disp = l1_sum[0] / denom
    loss_logits = ce_sum[0] / denom
    objective = WD * loss_disp + WL * loss_logits
    return objective, loss_disp, loss_logits
